# baseline (speedup 1.0000x reference)
import jax
import jax.numpy as jnp
from jax import lax
from jax.experimental import pallas as pl
from jax.experimental.pallas import tpu as pltpu
from jax.experimental.pallas import tpu_sc as plsc

N = 10000
E = 320000
D = 128
G = 64
OUT = 128

NC = 2
NS = 16
NW = NC * NS
C = 64
BATCH = 8
NCHUNK = BATCH * (-(-E // (NW * C * BATCH)))
NB = NCHUNK // BATCH
EW = NCHUNK * C
EPAD = NW * EW
NP = 10240
RPS = NP // NS
DW = D // 2
INV_SQRT_D = 1.0 / float(D) ** 0.5
HMASK = -65536


def _edge_body(q_hbm, kv_hbm, src_hbm, dst_hbm, outv_hbm, outd_hbm,
               srcb, dstb, qrows, kvrows, orows, denomv, acc_sh, sem):
    c_id = lax.axis_index("c")
    s_id = lax.axis_index("s")
    zeros16 = jnp.zeros((16,), jnp.float32)
    lanes = lax.broadcasted_iota(jnp.int32, (16,), 0)

    def _f32(w):
        return lax.bitcast_convert_type(w, jnp.float32)

    def _zero_orow(r, _):
        for t in range(D // 16):
            orows[r, pl.ds(16 * t, 16)] = zeros16
        return 0

    lax.fori_loop(0, C, _zero_orow, 0)
    row0 = s_id * RPS
    for i in range(RPS // C):
        pltpu.sync_copy(orows, acc_sh.at[pl.ds(row0 + i * C, C)])

    def _zero_denom(r, _):
        denomv[pl.ds(r * 16, 16)] = zeros16
        return 0

    lax.fori_loop(0, NP // 16, _zero_denom, 0)
    plsc.subcore_barrier()

    wid = c_id * NS + s_id

    def _batch(bi, _):
        bbase = pl.multiple_of(wid * EW + bi * (BATCH * C), 8)
        pltpu.sync_copy(src_hbm.at[pl.ds(bbase, BATCH * C)], srcb)
        pltpu.sync_copy(dst_hbm.at[pl.ds(bbase, BATCH * C)], dstb)

        def _chunk(ci, _):
            coff = pl.multiple_of(ci * C, 8)
            srcc = srcb.at[pl.ds(coff, C)]
            dstc = dstb.at[pl.ds(coff, C)]
            pltpu.async_copy(q_hbm.at[dstc], qrows, sem)
            pltpu.async_copy(kv_hbm.at[srcc], kvrows, sem)
            pltpu.make_async_copy(q_hbm.at[dstc], qrows, sem).wait()
            pltpu.make_async_copy(kv_hbm.at[srcc], kvrows, sem).wait()
            for g in range(C // 16):
                e0 = g * 16
                alpha = zeros16
                for j in range(16):
                    e = e0 + j
                    p = zeros16
                    for t in range(DW // 16):
                        kw = kvrows[e, pl.ds(16 * t, 16)]
                        p = p + (qrows[e, pl.ds(16 * t, 16)]
                                 * _f32(kw << 16))
                        p = p + (qrows[e, pl.ds(DW + 16 * t, 16)]
                                 * _f32(kw & HMASK))
                    alpha = jnp.where(lanes == j, jnp.sum(p), alpha)
                ex = jnp.exp(alpha * INV_SQRT_D)
                dst16 = dstb[pl.ds(coff + e0, 16)]
                plsc.addupdate_scatter(denomv, [dst16], ex)
                for j in range(16):
                    e = e0 + j
                    s = ex[j]
                    for t in range(DW // 16):
                        vw = kvrows[e, pl.ds(DW + 16 * t, 16)]
                        orows[e, pl.ds(16 * t, 16)] = _f32(vw << 16) * s
                        orows[e, pl.ds(DW + 16 * t, 16)] = (
                            _f32(vw & HMASK) * s)
            pltpu.sync_copy(orows, acc_sh.at[dstc], add=True)
            return 0

        lax.fori_loop(0, BATCH, _chunk, 0)
        return 0

    lax.fori_loop(0, NB, _batch, 0)
    plsc.subcore_barrier()

    wid8 = pl.multiple_of(wid * NP, 8)
    pltpu.sync_copy(denomv, outd_hbm.at[pl.ds(wid8, NP)])
    for i in range(RPS // C):
        r = row0 + i * C
        pltpu.sync_copy(acc_sh.at[pl.ds(r, C)], orows)
        pltpu.sync_copy(orows, outv_hbm.at[c_id, pl.ds(r, C)])


_edge_call = pl.kernel(
    _edge_body,
    out_type=(jax.ShapeDtypeStruct((NC, NP, D), jnp.float32),
              jax.ShapeDtypeStruct((NW * NP,), jnp.float32)),
    mesh=plsc.VectorSubcoreMesh(core_axis_name="c", subcore_axis_name="s"),
    compiler_params=pltpu.CompilerParams(needs_layout_passes=False),
    scratch_types=[
        pltpu.VMEM((BATCH * C,), jnp.int32),
        pltpu.VMEM((BATCH * C,), jnp.int32),
        pltpu.VMEM((C, D), jnp.float32),
        pltpu.VMEM((C, D), jnp.int32),
        pltpu.VMEM((C, D), jnp.float32),
        pltpu.VMEM((NP,), jnp.float32),
        pltpu.VMEM_SHARED((NP, D), jnp.float32),
        pltpu.SemaphoreType.DMA,
    ],
)


def _pack_bf16(a):
    u = lax.bitcast_convert_type(a, jnp.uint32)
    lo = jnp.right_shift(u[:, :DW] + jnp.uint32(0x8000), jnp.uint32(16))
    hi = (u[:, DW:] + jnp.uint32(0x8000)) & jnp.uint32(0xFFFF0000)
    return lax.bitcast_convert_type(lo | hi, jnp.int32)


def _qkv_body(x_ref, w_ref, b_ref, q_ref, kv_ref):
    y = jnp.dot(x_ref[...], w_ref[...], preferred_element_type=jnp.float32)
    y = y + b_ref[...]
    q_ref[...] = y[:, :D]
    kv_ref[...] = jnp.concatenate(
        [_pack_bf16(y[:, D:2 * D]), _pack_bf16(y[:, 2 * D:3 * D])], axis=1)


_qkv_call = pl.pallas_call(
    _qkv_body,
    out_shape=[jax.ShapeDtypeStruct((NP, D), jnp.float32),
               jax.ShapeDtypeStruct((NP, D), jnp.int32)],
)


def _node_h(acc_ref, dd_ref):
    a = acc_ref[0] + acc_ref[1]
    ones = jnp.ones((NW, 1), jnp.float32)
    den = lax.dot_general(dd_ref[...], ones, (((0,), (0,)), ((), ())),
                          preferred_element_type=jnp.float32)
    return jnp.maximum(a / (den + 1e-30), 0.0)


def _mid_body(acc_ref, dd_ref, w_ref, b_ref, q_ref, kv_ref):
    h = _node_h(acc_ref, dd_ref)
    y = jnp.dot(h, w_ref[...], preferred_element_type=jnp.float32)
    y = y + b_ref[...]
    q_ref[...] = y[:, :D]
    kv_ref[...] = jnp.concatenate(
        [_pack_bf16(y[:, D:2 * D]), _pack_bf16(y[:, 2 * D:3 * D])], axis=1)


_mid_call = pl.pallas_call(
    _mid_body,
    out_shape=[jax.ShapeDtypeStruct((NP, D), jnp.float32),
               jax.ShapeDtypeStruct((NP, D), jnp.int32)],
)


def _final_body(acc_ref, dd_ref, batch_ref, w1_ref, b1_ref, w2_ref, b2_ref,
                o_ref):
    h = _node_h(acc_ref, dd_ref)[:N, :]
    gid = lax.broadcasted_iota(jnp.int32, (1, G), 1)
    p = (batch_ref[...] == gid).astype(jnp.float32)
    g = lax.dot_general(p, h, (((0,), (0,)), ((), ())),
                        preferred_element_type=jnp.float32)
    z = jnp.maximum(
        jnp.dot(g, w1_ref[...], preferred_element_type=jnp.float32)
        + b1_ref[...], 0.0)
    o_ref[...] = (jnp.dot(z, w2_ref[...], preferred_element_type=jnp.float32)
                  + b2_ref[...])


_final_call = pl.pallas_call(
    _final_body,
    out_shape=jax.ShapeDtypeStruct((G, OUT), jnp.float32),
)


def kernel(x, edge_index, batch, Wq0, bq0, Wk0, bk0, Wv0, bv0,
           Wq1, bq1, Wk1, bk1, Wv1, bv1, lin1_W, lin1_b, lin2_W, lin2_b):
    src = jnp.concatenate(
        [edge_index[0], jnp.zeros((EPAD - E,), jnp.int32)])
    dst = jnp.concatenate(
        [edge_index[1], jnp.full((EPAD - E,), NP - 1, jnp.int32)])
    xp = jnp.concatenate([x, jnp.zeros((NP - N, D), jnp.float32)])
    w0 = jnp.concatenate([Wq0, Wk0, Wv0], axis=1)
    b0 = jnp.concatenate([bq0, bk0, bv0]).reshape(1, 3 * D)
    w1 = jnp.concatenate([Wq1, Wk1, Wv1], axis=1)
    b1 = jnp.concatenate([bq1, bk1, bv1]).reshape(1, 3 * D)

    q0, kv0 = _qkv_call(xp, w0, b0)
    acc0, dd0 = _edge_call(q0, kv0, src, dst)
    q1, kv1 = _mid_call(acc0, dd0.reshape(NW, NP), w1, b1)
    acc1, dd1 = _edge_call(q1, kv1, src, dst)
    out = _final_call(acc1, dd1.reshape(NW, NP), batch.reshape(N, 1), lin1_W,
                      lin1_b.reshape(1, D), lin2_W, lin2_b.reshape(1, OUT))
    return out


# speedup vs baseline: 1.5458x; 1.5458x over previous
import jax
import jax.numpy as jnp
from jax import lax
from jax.experimental import pallas as pl
from jax.experimental.pallas import tpu as pltpu
from jax.experimental.pallas import tpu_sc as plsc

N = 10000
E = 320000
D = 128
G = 64
OUT = 128

NC = 2
NS = 16
NW = NC * NS
C = 48
NCHUNK = 2 * (-(-E // (NW * C * 2)))
NI = NCHUNK // 2
EW = NCHUNK * C
EPAD = NW * EW
NP = 10240
RPS = NP // NS
DW = D // 2
INV_SQRT_D = 1.0 / float(D) ** 0.5
HMASK = -65536


def _edge_body(q_hbm, kv_hbm, src_hbm, dst_hbm, zeros_hbm, outv_hbm,
               outd_hbm, srcb0, dstb0, qrows0, kvrows0,
               srcb1, dstb1, qrows1, kvrows1,
               orows, denomv, acc_sh, sem0, sem1):
    c_id = lax.axis_index("c")
    s_id = lax.axis_index("s")
    zeros16 = jnp.zeros((16,), jnp.float32)
    lanes = lax.broadcasted_iota(jnp.int32, (16,), 0)

    def _f32(w):
        return lax.bitcast_convert_type(w, jnp.float32)

    row0 = pl.multiple_of(s_id * RPS, 8)
    pltpu.sync_copy(zeros_hbm.at[pl.ds(row0, RPS)],
                    acc_sh.at[pl.ds(row0, RPS)])

    def _zero_denom(r, _):
        denomv[pl.ds(r * 16, 16)] = zeros16
        return 0

    lax.fori_loop(0, NP // 16, _zero_denom, 0)
    plsc.subcore_barrier()

    wid = c_id * NS + s_id

    def _stage(ci, srcb, dstb, qrows, kvrows, sem):
        base = pl.multiple_of(wid * EW + ci * C, 8)
        pltpu.sync_copy(src_hbm.at[pl.ds(base, C)], srcb)
        pltpu.sync_copy(dst_hbm.at[pl.ds(base, C)], dstb)
        pltpu.async_copy(q_hbm.at[dstb], qrows, sem)
        pltpu.async_copy(kv_hbm.at[srcb], kvrows, sem)

    def _consume(srcb, dstb, qrows, kvrows, sem):
        pltpu.make_async_copy(q_hbm.at[dstb], qrows, sem).wait()
        pltpu.make_async_copy(kv_hbm.at[srcb], kvrows, sem).wait()
        for g in range(C // 16):
            e0 = g * 16
            alpha = zeros16
            for j in range(16):
                e = e0 + j
                p = zeros16
                for t in range(DW // 16):
                    kw = kvrows[e, pl.ds(16 * t, 16)]
                    p = p + (qrows[e, pl.ds(16 * t, 16)] * _f32(kw << 16))
                    p = p + (qrows[e, pl.ds(DW + 16 * t, 16)]
                             * _f32(kw & HMASK))
                alpha = jnp.where(lanes == j, jnp.sum(p), alpha)
            ex = jnp.exp(alpha * INV_SQRT_D)
            dst16 = dstb[pl.ds(e0, 16)]
            plsc.addupdate_scatter(denomv, [dst16], ex)
            for j in range(16):
                e = e0 + j
                s = ex[j]
                for t in range(DW // 16):
                    vw = kvrows[e, pl.ds(DW + 16 * t, 16)]
                    orows[e, pl.ds(16 * t, 16)] = _f32(vw << 16) * s
                    orows[e, pl.ds(DW + 16 * t, 16)] = _f32(vw & HMASK) * s
        pltpu.sync_copy(orows, acc_sh.at[dstb], add=True)

    set0 = (srcb0, dstb0, qrows0, kvrows0, sem0)
    set1 = (srcb1, dstb1, qrows1, kvrows1, sem1)
    _stage(0, *set0)

    def _pipe(i, _):
        _stage(2 * i + 1, *set1)
        _consume(*set0)

        @pl.when(i < NI - 1)
        def _():
            _stage(2 * i + 2, *set0)

        _consume(*set1)
        return 0

    lax.fori_loop(0, NI, _pipe, 0)
    plsc.subcore_barrier()

    wid8 = pl.multiple_of(wid * NP, 8)
    pltpu.sync_copy(denomv, outd_hbm.at[pl.ds(wid8, NP)])
    pltpu.sync_copy(acc_sh.at[pl.ds(row0, RPS)],
                    outv_hbm.at[c_id, pl.ds(row0, RPS)])


_edge_call = pl.kernel(
    _edge_body,
    out_type=(jax.ShapeDtypeStruct((NC, NP, D), jnp.float32),
              jax.ShapeDtypeStruct((NW * NP,), jnp.float32)),
    mesh=plsc.VectorSubcoreMesh(core_axis_name="c", subcore_axis_name="s"),
    compiler_params=pltpu.CompilerParams(needs_layout_passes=False),
    scratch_types=[
        pltpu.VMEM((C,), jnp.int32),
        pltpu.VMEM((C,), jnp.int32),
        pltpu.VMEM((C, D), jnp.float32),
        pltpu.VMEM((C, D), jnp.int32),
        pltpu.VMEM((C,), jnp.int32),
        pltpu.VMEM((C,), jnp.int32),
        pltpu.VMEM((C, D), jnp.float32),
        pltpu.VMEM((C, D), jnp.int32),
        pltpu.VMEM((C, D), jnp.float32),
        pltpu.VMEM((NP,), jnp.float32),
        pltpu.VMEM_SHARED((NP, D), jnp.float32),
        pltpu.SemaphoreType.DMA,
        pltpu.SemaphoreType.DMA,
    ],
)


def _pack_bf16(a):
    u = lax.bitcast_convert_type(a, jnp.uint32)
    lo = jnp.right_shift(u[:, :DW] + jnp.uint32(0x8000), jnp.uint32(16))
    hi = (u[:, DW:] + jnp.uint32(0x8000)) & jnp.uint32(0xFFFF0000)
    return lax.bitcast_convert_type(lo | hi, jnp.int32)


def _qkv_body(x_ref, w_ref, b_ref, q_ref, kv_ref):
    y = jnp.dot(x_ref[...], w_ref[...], preferred_element_type=jnp.float32)
    y = y + b_ref[...]
    q_ref[...] = y[:, :D]
    kv_ref[...] = jnp.concatenate(
        [_pack_bf16(y[:, D:2 * D]), _pack_bf16(y[:, 2 * D:3 * D])], axis=1)


_qkv_call = pl.pallas_call(
    _qkv_body,
    out_shape=[jax.ShapeDtypeStruct((NP, D), jnp.float32),
               jax.ShapeDtypeStruct((NP, D), jnp.int32)],
)


def _node_h(acc_ref, dd_ref):
    a = acc_ref[0] + acc_ref[1]
    ones = jnp.ones((NW, 1), jnp.float32)
    den = lax.dot_general(dd_ref[...], ones, (((0,), (0,)), ((), ())),
                          preferred_element_type=jnp.float32)
    return jnp.maximum(a / (den + 1e-30), 0.0)


def _mid_body(acc_ref, dd_ref, w_ref, b_ref, q_ref, kv_ref):
    h = _node_h(acc_ref, dd_ref)
    y = jnp.dot(h, w_ref[...], preferred_element_type=jnp.float32)
    y = y + b_ref[...]
    q_ref[...] = y[:, :D]
    kv_ref[...] = jnp.concatenate(
        [_pack_bf16(y[:, D:2 * D]), _pack_bf16(y[:, 2 * D:3 * D])], axis=1)


_mid_call = pl.pallas_call(
    _mid_body,
    out_shape=[jax.ShapeDtypeStruct((NP, D), jnp.float32),
               jax.ShapeDtypeStruct((NP, D), jnp.int32)],
)


def _final_body(acc_ref, dd_ref, batch_ref, w1_ref, b1_ref, w2_ref, b2_ref,
                o_ref):
    h = _node_h(acc_ref, dd_ref)[:N, :]
    gid = lax.broadcasted_iota(jnp.int32, (1, G), 1)
    p = (batch_ref[...] == gid).astype(jnp.float32)
    g = lax.dot_general(p, h, (((0,), (0,)), ((), ())),
                        preferred_element_type=jnp.float32)
    z = jnp.maximum(
        jnp.dot(g, w1_ref[...], preferred_element_type=jnp.float32)
        + b1_ref[...], 0.0)
    o_ref[...] = (jnp.dot(z, w2_ref[...], preferred_element_type=jnp.float32)
                  + b2_ref[...])


_final_call = pl.pallas_call(
    _final_body,
    out_shape=jax.ShapeDtypeStruct((G, OUT), jnp.float32),
)


def kernel(x, edge_index, batch, Wq0, bq0, Wk0, bk0, Wv0, bv0,
           Wq1, bq1, Wk1, bk1, Wv1, bv1, lin1_W, lin1_b, lin2_W, lin2_b):
    src = jnp.concatenate(
        [edge_index[0], jnp.zeros((EPAD - E,), jnp.int32)])
    dst = jnp.concatenate(
        [edge_index[1], jnp.full((EPAD - E,), NP - 1, jnp.int32)])
    xp = jnp.concatenate([x, jnp.zeros((NP - N, D), jnp.float32)])
    w0 = jnp.concatenate([Wq0, Wk0, Wv0], axis=1)
    b0 = jnp.concatenate([bq0, bk0, bv0]).reshape(1, 3 * D)
    w1 = jnp.concatenate([Wq1, Wk1, Wv1], axis=1)
    b1 = jnp.concatenate([bq1, bk1, bv1]).reshape(1, 3 * D)

    zrows = jnp.zeros((NP, D), jnp.float32)
    q0, kv0 = _qkv_call(xp, w0, b0)
    acc0, dd0 = _edge_call(q0, kv0, src, dst, zrows)
    q1, kv1 = _mid_call(acc0, dd0.reshape(NW, NP), w1, b1)
    acc1, dd1 = _edge_call(q1, kv1, src, dst, zrows)
    out = _final_call(acc1, dd1.reshape(NW, NP), batch.reshape(N, 1), lin1_W,
                      lin1_b.reshape(1, D), lin2_W, lin2_b.reshape(1, OUT))
    return out
